# baseline (speedup 1.0000x reference)
import jax
import jax.numpy as jnp
from jax import lax
from jax.experimental import pallas as pl

_C = 81
_TH = 0.5
_RATIO = 3
_V0 = 0.1
_V1 = 0.2
_FINF_BITS = 0x7F800000


def _fused_body(tgt_ref, pt_ref, ld_ref, c_ref,
                lossl_ref, lossc_ref, npos_ref):
    tgt = tgt_ref[0]
    A = tgt.shape[0]
    P = pt_ref.shape[1]

    gx1 = tgt[:, 0:1]
    gy1 = tgt[:, 1:2]
    gx2 = tgt[:, 2:3]
    gy2 = tgt[:, 3:4]
    labels = tgt[:, 10:11]
    pcx = pt_ref[0:1, :]
    pcy = pt_ref[1:2, :]
    pw = pt_ref[2:3, :]
    ph = pt_ref[3:4, :]
    px1 = pcx - pw / 2
    py1 = pcy - ph / 2
    px2 = pcx + pw / 2
    py2 = pcy + ph / 2

    iw = jnp.maximum(jnp.minimum(gx2, px2) - jnp.maximum(gx1, px1), 0.0)
    ih = jnp.maximum(jnp.minimum(gy2, py2) - jnp.maximum(gy1, py1), 0.0)
    inter = iw * ih
    area_a = (gx2 - gx1) * (gy2 - gy1)
    area_b = (px2 - px1) * (py2 - py1)
    ov = inter / (area_a + area_b - inter)

    ja = lax.broadcasted_iota(jnp.int32, (A, 1), 0)
    ip = lax.broadcasted_iota(jnp.int32, (1, P), 1)

    bto0 = jnp.max(ov, axis=0, keepdims=True)
    bti0 = jnp.min(jnp.where(ov == bto0, ja, A), axis=0, keepdims=True)
    rowm = jnp.max(ov, axis=1, keepdims=True)
    bpi = jnp.min(jnp.where(ov == rowm, ip, P), axis=1, keepdims=True)

    eq = ip == bpi
    any_eq = jnp.max(jnp.where(eq, 1, 0), axis=0, keepdims=True) > 0
    btj = jnp.max(jnp.where(eq, ja, -1), axis=0, keepdims=True)
    bto = jnp.where(any_eq, 2.0, bto0)
    bti = jnp.where(any_eq, btj, bti0)

    oh = jnp.where(bti == ja, 1.0, 0.0)
    lab = jnp.sum(oh * labels, axis=0, keepdims=True)
    pos = bto >= _TH
    ct = jnp.where(pos, lab + 1.0, 0.0).astype(jnp.int32)

    def mgat(c):
        return jnp.sum(oh * tgt[:, c:c + 1], axis=0, keepdims=True)

    m0, m1, m2, m3 = mgat(0), mgat(1), mgat(2), mgat(3)
    m4, m5, m6, m7 = mgat(4), mgat(5), mgat(6), mgat(7)
    m8, m9 = mgat(8), mgat(9)

    enc = (
        ((m0 + m2) / 2 - pcx) / (_V0 * pw),
        ((m1 + m3) / 2 - pcy) / (_V0 * ph),
        jnp.log((m2 - m0) / pw) / _V1,
        jnp.log((m3 - m1) / ph) / _V1,
        jnp.log(m4 / pw + 0.1) / _V1,
        jnp.log(m5 / ph + 0.1) / _V1,
        jnp.log(m6 / pw + 0.1) / _V1,
        jnp.log(m7 / ph + 0.1) / _V1,
        (m8 - pcx) / (_V0 * pw),
        (m9 - pcy) / (_V0 * ph),
    )
    ld = ld_ref[0].T
    lossl = jnp.zeros((), jnp.float32)
    for c in range(10):
        d = ld[c:c + 1, :] - enc[c]
        ad = jnp.abs(d)
        sl1 = jnp.where(ad < 1.0, 0.5 * d * d, ad - 0.5)
        lossl = lossl + jnp.sum(jnp.where(pos, sl1, 0.0))

    xt = c_ref[0].T
    m = jnp.max(xt, axis=0, keepdims=True)
    s = jnp.sum(jnp.exp(xt - m), axis=0, keepdims=True)
    lse = jnp.log(s) + m
    ic = lax.broadcasted_iota(jnp.int32, (_C, 1), 0)
    g = jnp.sum(jnp.where(ic == ct, xt, 0.0), axis=0, keepdims=True)
    ce = lse - g

    v = jnp.where(pos, 0.0, ce)
    vi = lax.bitcast_convert_type(v, jnp.int32)
    npos = jnp.sum(jnp.where(pos, 1, 0))
    k = jnp.minimum(_RATIO * npos, P - 1)

    def cnt_ge(t):
        return jnp.sum(jnp.where(vi >= t, 1, 0))

    def bs_val(_, lh):
        lo, hi = lh
        mid = lo + (hi - lo) // 2
        take = cnt_ge(mid) >= k
        return jnp.where(take, mid, lo), jnp.where(take, hi, mid)

    t, _ = lax.fori_loop(
        0, 31, bs_val, (jnp.int32(0), jnp.int32(_FINF_BITS)))

    krem = k - cnt_ge(t + 1)
    eqm = vi == t

    def cnt_lt(mm):
        return jnp.sum(jnp.where(eqm & (ip < mm), 1, 0))

    def bs_idx(_, lh):
        lo, hi = lh
        mid = lo + (hi - lo) // 2
        take = cnt_lt(mid) >= krem
        return jnp.where(take, lo, mid), jnp.where(take, mid, hi)

    _, mstar = lax.fori_loop(0, 14, bs_idx, (jnp.int32(0), jnp.int32(P)))

    sel = pos | (vi > t) | (eqm & (ip < mstar))
    lossc = jnp.sum(jnp.where(sel, ce, 0.0))

    lossl_ref[...] = jnp.full((1, 1, 1), lossl, jnp.float32)
    lossc_ref[...] = jnp.full((1, 1, 1), lossc, jnp.float32)
    npos_ref[...] = jnp.full((1, 1, 1), npos, jnp.int32)


def kernel(loc_data, conf_data, priors, targets):
    B, P, _ = loc_data.shape
    A = targets.shape[1]
    p_t = jnp.transpose(priors, (1, 0))

    lossl, lossc, npos = pl.pallas_call(
        _fused_body,
        grid=(B,),
        in_specs=[
            pl.BlockSpec((1, A, 11), lambda b: (b, 0, 0)),
            pl.BlockSpec((4, P), lambda b: (0, 0)),
            pl.BlockSpec((1, P, 10), lambda b: (b, 0, 0)),
            pl.BlockSpec((1, P, _C), lambda b: (b, 0, 0)),
        ],
        out_specs=[
            pl.BlockSpec((1, 1, 1), lambda b: (b, 0, 0)),
            pl.BlockSpec((1, 1, 1), lambda b: (b, 0, 0)),
            pl.BlockSpec((1, 1, 1), lambda b: (b, 0, 0)),
        ],
        out_shape=[
            jax.ShapeDtypeStruct((B, 1, 1), jnp.float32),
            jax.ShapeDtypeStruct((B, 1, 1), jnp.float32),
            jax.ShapeDtypeStruct((B, 1, 1), jnp.int32),
        ],
    )(targets, p_t, loc_data, conf_data)

    n = jnp.maximum(jnp.sum(npos).astype(jnp.float32), 1.0)
    return jnp.sum(lossl) / n, jnp.sum(lossc) / n


# speedup vs baseline: 1.8185x; 1.8185x over previous
import jax
import jax.numpy as jnp
from jax import lax
from jax.experimental import pallas as pl

_C = 81
_TH = 0.5
_RATIO = 3
_V0 = 0.1
_V1 = 0.2
_FINF_BITS = 0x7F800000
_G = 4


def _fused_body(tgt_ref, pt_ref, ld_ref, c_ref,
                lossl_ref, lossc_ref, npos_ref):
    A = tgt_ref.shape[1]
    P = pt_ref.shape[1]

    pcx = pt_ref[0:1, :]
    pcy = pt_ref[1:2, :]
    pw = pt_ref[2:3, :]
    ph = pt_ref[3:4, :]
    px1 = pcx - pw / 2
    py1 = pcy - ph / 2
    px2 = pcx + pw / 2
    py2 = pcy + ph / 2
    area_b = (px2 - px1) * (py2 - py1)

    ja = lax.broadcasted_iota(jnp.int32, (A, 1), 0)
    ip = lax.broadcasted_iota(jnp.int32, (1, P), 1)

    lossl = jnp.zeros((), jnp.float32)
    pos_rows = []
    ce_rows = []
    np_cols = []

    for g in range(_G):
        tgt = tgt_ref[g]
        gx1 = tgt[:, 0:1]
        gy1 = tgt[:, 1:2]
        gx2 = tgt[:, 2:3]
        gy2 = tgt[:, 3:4]
        labels = tgt[:, 10:11]

        iw = jnp.maximum(jnp.minimum(gx2, px2) - jnp.maximum(gx1, px1), 0.0)
        ih = jnp.maximum(jnp.minimum(gy2, py2) - jnp.maximum(gy1, py1), 0.0)
        inter = iw * ih
        area_a = (gx2 - gx1) * (gy2 - gy1)
        ov = inter / (area_a + area_b - inter)

        bto0 = jnp.max(ov, axis=0, keepdims=True)
        bti0 = jnp.min(jnp.where(ov == bto0, ja, A), axis=0, keepdims=True)
        rowm = jnp.max(ov, axis=1, keepdims=True)
        bpi = jnp.min(jnp.where(ov == rowm, ip, P), axis=1, keepdims=True)

        eq = ip == bpi
        any_eq = jnp.max(jnp.where(eq, 1, 0), axis=0, keepdims=True) > 0
        btj = jnp.max(jnp.where(eq, ja, -1), axis=0, keepdims=True)
        bto = jnp.where(any_eq, 2.0, bto0)
        bti = jnp.where(any_eq, btj, bti0)

        ohf = jnp.where(bti == ja, 1.0, 0.0)
        lab = jnp.sum(ohf * labels, axis=0, keepdims=True)
        pos = bto >= _TH
        ct = jnp.where(pos, lab + 1.0, 0.0).astype(jnp.int32)

        truths_t = tgt[:, 0:10].T
        mt = jax.lax.dot_general(
            truths_t, ohf, (((1,), (0,)), ((), ())),
            precision=lax.Precision.HIGHEST,
            preferred_element_type=jnp.float32)
        m0, m1, m2, m3 = (mt[c:c + 1, :] for c in range(0, 4))
        m4, m5, m6, m7 = (mt[c:c + 1, :] for c in range(4, 8))
        m8, m9 = mt[8:9, :], mt[9:10, :]

        enc = jnp.concatenate([
            ((m0 + m2) / 2 - pcx) / (_V0 * pw),
            ((m1 + m3) / 2 - pcy) / (_V0 * ph),
            jnp.log((m2 - m0) / pw) / _V1,
            jnp.log((m3 - m1) / ph) / _V1,
            jnp.log(m4 / pw + 0.1) / _V1,
            jnp.log(m5 / ph + 0.1) / _V1,
            jnp.log(m6 / pw + 0.1) / _V1,
            jnp.log(m7 / ph + 0.1) / _V1,
            (m8 - pcx) / (_V0 * pw),
            (m9 - pcy) / (_V0 * ph),
        ], axis=0)
        d = ld_ref[g] - enc
        ad = jnp.abs(d)
        sl1 = jnp.where(ad < 1.0, 0.5 * d * d, ad - 0.5)
        lossl = lossl + jnp.sum(jnp.where(pos, sl1, 0.0))

        xt = c_ref[g].T
        m = jnp.max(xt, axis=0, keepdims=True)
        s = jnp.sum(jnp.exp(xt - m), axis=0, keepdims=True)
        lse = jnp.log(s) + m
        ic = lax.broadcasted_iota(jnp.int32, (_C, 1), 0)
        gat = jnp.sum(jnp.where(ic == ct, xt, 0.0), axis=0, keepdims=True)
        ce = lse - gat

        pos_rows.append(jnp.where(pos, 1, 0))
        ce_rows.append(ce)
        np_cols.append(jnp.full((1, 1), jnp.sum(jnp.where(pos, 1, 0)),
                                jnp.int32))

    posg = jnp.concatenate(pos_rows, axis=0) > 0
    ceg = jnp.concatenate(ce_rows, axis=0)
    npg = jnp.concatenate(np_cols, axis=0)
    v = jnp.where(posg, 0.0, ceg)
    vi = lax.bitcast_convert_type(v, jnp.int32)
    k = jnp.minimum(_RATIO * npg, P - 1)

    def cnt_ge(t):
        return jnp.sum(jnp.where(vi >= t, 1, 0), axis=1, keepdims=True)

    def bs_val(_, lh):
        lo, hi = lh
        mid = lo + (hi - lo) // 2
        take = cnt_ge(mid) >= k
        return jnp.where(take, mid, lo), jnp.where(take, hi, mid)

    t, _ = lax.fori_loop(
        0, 31, bs_val,
        (jnp.zeros((_G, 1), jnp.int32),
         jnp.full((_G, 1), _FINF_BITS, jnp.int32)))

    krem = k - cnt_ge(t + 1)
    eqm = vi == t

    def cnt_lt(mm):
        return jnp.sum(jnp.where(eqm & (ip < mm), 1, 0),
                       axis=1, keepdims=True)

    def bs_idx(_, lh):
        lo, hi = lh
        mid = lo + (hi - lo) // 2
        take = cnt_lt(mid) >= krem
        return jnp.where(take, lo, mid), jnp.where(take, mid, hi)

    _, mstar = lax.fori_loop(
        0, 14, bs_idx,
        (jnp.zeros((_G, 1), jnp.int32), jnp.full((_G, 1), P, jnp.int32)))

    sel = posg | (vi > t) | (eqm & (ip < mstar))
    lossc = jnp.sum(jnp.where(sel, ceg, 0.0))
    npos = jnp.sum(npg)

    @pl.when(pl.program_id(0) == 0)
    def _():
        lossl_ref[...] = jnp.zeros((1, 1), jnp.float32)
        lossc_ref[...] = jnp.zeros((1, 1), jnp.float32)
        npos_ref[...] = jnp.zeros((1, 1), jnp.int32)

    lossl_ref[...] += jnp.full((1, 1), lossl, jnp.float32)
    lossc_ref[...] += jnp.full((1, 1), lossc, jnp.float32)
    npos_ref[...] += jnp.full((1, 1), npos, jnp.int32)


def kernel(loc_data, conf_data, priors, targets):
    B, P, _ = loc_data.shape
    A = targets.shape[1]
    ld_t = jnp.transpose(loc_data, (0, 2, 1))
    p_t = jnp.transpose(priors, (1, 0))

    lossl, lossc, npos = pl.pallas_call(
        _fused_body,
        grid=(B // _G,),
        in_specs=[
            pl.BlockSpec((_G, A, 11), lambda b: (b, 0, 0)),
            pl.BlockSpec((4, P), lambda b: (0, 0)),
            pl.BlockSpec((_G, 10, P), lambda b: (b, 0, 0)),
            pl.BlockSpec((_G, P, _C), lambda b: (b, 0, 0)),
        ],
        out_specs=[
            pl.BlockSpec((1, 1), lambda b: (0, 0)),
            pl.BlockSpec((1, 1), lambda b: (0, 0)),
            pl.BlockSpec((1, 1), lambda b: (0, 0)),
        ],
        out_shape=[
            jax.ShapeDtypeStruct((1, 1), jnp.float32),
            jax.ShapeDtypeStruct((1, 1), jnp.float32),
            jax.ShapeDtypeStruct((1, 1), jnp.int32),
        ],
    )(targets, p_t, ld_t, conf_data)

    n = jnp.maximum(npos[0, 0].astype(jnp.float32), 1.0)
    return lossl[0, 0] / n, lossc[0, 0] / n
